# baseline (speedup 1.0000x reference)
import functools

import jax
import jax.numpy as jnp
from jax import lax
from jax.experimental import pallas as pl
from jax.experimental.pallas import tpu as pltpu
from jax.experimental.pallas import tpu_sc as plsc

B = 1024
N = 3129
D = 768
NP = 3200
RB = 128
NB = NP // RB
NEG = -3.0e38

NC = 2
NS = 16
NW = NC * NS
ROWS_W = B // NW
CH = 640
NCH = NP // CH


def _a2_body(e_blk, e_full, map_ref, rec_ref):
    i = pl.program_id(0)

    @pl.when(i == 0)
    def _():
        e = e_full[...]
        g = lax.dot_general(e, e, (((0,), (0,)), ((), ())),
                            preferred_element_type=jnp.float32)
        h = lax.dot_general(e, g.astype(jnp.bfloat16),
                            (((1,), (0,)), ((), ())),
                            preferred_element_type=jnp.float32)
        n2 = jnp.sum(e.astype(jnp.float32) * h, axis=1)
        rec_ref[...] = (1.0 / jnp.sqrt(n2)).reshape(1, NP)

    s = lax.dot_general(e_blk[...], e_full[...], (((1,), (1,)), ((), ())),
                        preferred_element_type=jnp.float32)
    ref_ = rec_ref[...].reshape(NP)
    reb = rec_ref[0, pl.ds(i * RB, RB)]
    v = s * (reb[:, None] * ref_[None, :])
    col = lax.broadcasted_iota(jnp.int32, (RB, NP), 1)
    rowg = i * RB + lax.broadcasted_iota(jnp.int32, (RB, NP), 0)
    v = jnp.where(col == rowg, -2.0, v)
    v = jnp.where(col >= N, NEG, v)
    m = jnp.max(v, axis=1, keepdims=True)
    idx = jnp.min(jnp.where(v == m, col, NP), axis=1)
    map_ref[...] = idx.astype(jnp.int32).reshape(1, 1, RB)


def _c_body(x_ref, wt_ref, out_ref):
    x = x_ref[...]
    f = jnp.maximum(x, 0.0) + jnp.log(1.0 + jnp.exp(-jnp.abs(x)))
    w = jnp.transpose(wt_ref[...])[:, :N]
    out_ref[...] = (jnp.sum(f) - jnp.sum(x * w)).reshape(1, 1, 1)


def _sc_body(lab_hbm, map_hbm, wint_hbm, map_v, lab_v, win_v, sem):
    wid = lax.axis_index("s") * NC + lax.axis_index("c")
    pltpu.sync_copy(map_hbm, map_v)
    iot = lax.iota(jnp.int32, 16)
    zeros16 = jnp.zeros((16,), jnp.float32)

    out_dma = None
    for grp in range(ROWS_W // 16):
        r0 = wid * ROWS_W + grp * 16
        pltpu.sync_copy(lab_hbm.at[pl.ds(r0, 16), :], lab_v)
        if out_dma is not None:
            out_dma.wait()

        def _initk(k, _):
            for kk in range(8):
                win_v[k * 8 + kk, :] = zeros16
            return 0
        lax.fori_loop(0, NP // 8, _initk, 0)

        UN = 8
        def _ph1(t, _):
            base = t * UN
            mch = map_v[pl.ds(base, 16)]
            for k in range(UN):
                gs = jnp.full((16,), base + k, jnp.int32)
                mvec = jnp.full((16,), mch[k], jnp.int32)
                lv = plsc.load_gather(lab_v, [iot, gs])
                plsc.store_scatter(win_v, [mvec, iot], lv, mask=lv != 0.0)
            return 0
        lax.fori_loop(0, N // UN, _ph1, 0)
        mch = map_v[pl.ds((N // UN) * UN, 16)]
        for k in range(N % UN):
            g = (N // UN) * UN + k
            gs = jnp.full((16,), g, jnp.int32)
            mvec = jnp.full((16,), mch[k], jnp.int32)
            lv = plsc.load_gather(lab_v, [iot, gs])
            plsc.store_scatter(win_v, [mvec, iot], lv, mask=lv != 0.0)

        out_dma = pltpu.async_copy(
            win_v, wint_hbm.at[:, pl.ds(r0, 16)], sem)
    out_dma.wait()


@jax.jit
def kernel(prediction, label, Embedding):
    ep = jnp.pad(Embedding, ((0, NP - N), (0, 0))).astype(jnp.bfloat16)
    mapping = pl.pallas_call(
        _a2_body,
        grid=(NB,),
        in_specs=[
            pl.BlockSpec((RB, D), lambda i: (i, 0)),
            pl.BlockSpec((NP, D), lambda i: (0, 0)),
        ],
        out_specs=pl.BlockSpec((1, 1, RB), lambda i: (i, 0, 0)),
        out_shape=jax.ShapeDtypeStruct((NB, 1, RB), jnp.int32),
        scratch_shapes=[pltpu.VMEM((1, NP), jnp.float32)],
    )(ep, ep).reshape(NP)

    mesh = plsc.VectorSubcoreMesh(core_axis_name="c", subcore_axis_name="s")
    wint = pl.kernel(
        _sc_body,
        out_type=jax.ShapeDtypeStruct((NP, B), jnp.float32),
        mesh=mesh,
        compiler_params=pltpu.CompilerParams(
            use_tc_tiling_on_sc=False, needs_layout_passes=False),
        scratch_types=[
            pltpu.VMEM((NP,), jnp.int32),
            pltpu.VMEM((16, N), jnp.float32),
            pltpu.VMEM((NP, 16), jnp.float32),
            pltpu.SemaphoreType.DMA,
        ],
    )(label, mapping)

    parts = pl.pallas_call(
        _c_body,
        grid=(B // RB,),
        in_specs=[
            pl.BlockSpec((RB, N), lambda i: (i, 0)),
            pl.BlockSpec((NP, RB), lambda i: (0, i)),
        ],
        out_specs=pl.BlockSpec((1, 1, 1), lambda i: (i, 0, 0)),
        out_shape=jax.ShapeDtypeStruct((B // RB, 1, 1), jnp.float32),
    )(prediction, wint)

    return jnp.sum(parts) / jnp.float32(B)


# speedup vs baseline: 1.3043x; 1.3043x over previous
import functools

import jax
import jax.numpy as jnp
from jax import lax
from jax.experimental import pallas as pl
from jax.experimental.pallas import tpu as pltpu
from jax.experimental.pallas import tpu_sc as plsc

B = 1024
N = 3129
D = 768
NP = 3200
RB = 128
NB = NP // RB
NEG = -3.0e38

NC = 2
NS = 16
NW = NC * NS
ROWS_W = B // NW
CH = 640
NCH = NP // CH


def _a2_body(e_blk, e_full, map_ref, rec_ref):
    i = pl.program_id(0)

    @pl.when(i == 0)
    def _():
        e = e_full[...]
        g = lax.dot_general(e, e, (((0,), (0,)), ((), ())),
                            preferred_element_type=jnp.float32)
        h = lax.dot_general(e, g.astype(jnp.bfloat16),
                            (((1,), (0,)), ((), ())),
                            preferred_element_type=jnp.float32)
        n2 = jnp.sum(e.astype(jnp.float32) * h, axis=1)
        rec_ref[...] = (1.0 / jnp.sqrt(n2)).reshape(1, NP)

    s = lax.dot_general(e_blk[...], e_full[...], (((1,), (1,)), ((), ())),
                        preferred_element_type=jnp.float32)
    ref_ = rec_ref[...].reshape(NP)
    reb = rec_ref[0, pl.ds(i * RB, RB)]
    v = s * (reb[:, None] * ref_[None, :])
    col = lax.broadcasted_iota(jnp.int32, (RB, NP), 1)
    rowg = i * RB + lax.broadcasted_iota(jnp.int32, (RB, NP), 0)
    v = jnp.where(col == rowg, -2.0, v)
    v = jnp.where(col >= N, NEG, v)
    m = jnp.max(v, axis=1, keepdims=True)
    idx = jnp.min(jnp.where(v == m, col, NP), axis=1)
    map_ref[...] = idx.astype(jnp.int32).reshape(1, 1, RB)


def _c_body(x_ref, wt_ref, out_ref):
    x = x_ref[...]
    f = jnp.maximum(x, 0.0) + jnp.log(1.0 + jnp.exp(-jnp.abs(x)))
    w = jnp.transpose(wt_ref[...])[:, :N]
    out_ref[...] = (jnp.sum(f) - jnp.sum(x * w)).reshape(1, 1, 1)


def _sc_body(labt_hbm, map_hbm, wint_hbm, map_v, lab_v, win_v, sem):
    wid = lax.axis_index("s") * NC + lax.axis_index("c")
    pltpu.sync_copy(map_hbm, map_v)
    iot = lax.iota(jnp.int32, 16)
    zeros16 = jnp.zeros((16,), jnp.float32)

    out_dma = None
    for grp in range(ROWS_W // 16):
        r0 = wid * ROWS_W + grp * 16
        pltpu.sync_copy(labt_hbm.at[:, pl.ds(r0, 16)], lab_v)
        if out_dma is not None:
            out_dma.wait()

        def _initk(k, _):
            for kk in range(8):
                win_v[k * 8 + kk, :] = zeros16
            return 0
        lax.fori_loop(0, NP // 8, _initk, 0)

        UN = 8
        def _ph1(t, _):
            base = t * UN
            mch = map_v[pl.ds(base, 16)]
            for k in range(UN):
                gs = jnp.full((16,), base + k, jnp.int32)
                mvec = jnp.full((16,), mch[k], jnp.int32)
                lv = plsc.load_gather(lab_v, [gs, iot])
                plsc.store_scatter(win_v, [mvec, iot], lv, mask=lv != 0.0)
            return 0
        lax.fori_loop(0, N // UN, _ph1, 0)
        mch = map_v[pl.ds((N // UN) * UN, 16)]
        for k in range(N % UN):
            g = (N // UN) * UN + k
            gs = jnp.full((16,), g, jnp.int32)
            mvec = jnp.full((16,), mch[k], jnp.int32)
            lv = plsc.load_gather(lab_v, [gs, iot])
            plsc.store_scatter(win_v, [mvec, iot], lv, mask=lv != 0.0)

        out_dma = pltpu.async_copy(
            win_v, wint_hbm.at[:, pl.ds(r0, 16)], sem)
    out_dma.wait()


@jax.jit
def kernel(prediction, label, Embedding):
    ep = jnp.pad(Embedding, ((0, NP - N), (0, 0))).astype(jnp.bfloat16)
    mapping = pl.pallas_call(
        _a2_body,
        grid=(NB,),
        in_specs=[
            pl.BlockSpec((RB, D), lambda i: (i, 0)),
            pl.BlockSpec((NP, D), lambda i: (0, 0)),
        ],
        out_specs=pl.BlockSpec((1, 1, RB), lambda i: (i, 0, 0)),
        out_shape=jax.ShapeDtypeStruct((NB, 1, RB), jnp.int32),
        scratch_shapes=[pltpu.VMEM((1, NP), jnp.float32)],
    )(ep, ep).reshape(NP)

    labelt = jnp.pad(label.T, ((0, NP - N), (0, 0)))

    mesh = plsc.VectorSubcoreMesh(core_axis_name="c", subcore_axis_name="s")
    wint = pl.kernel(
        _sc_body,
        out_type=jax.ShapeDtypeStruct((NP, B), jnp.float32),
        mesh=mesh,
        compiler_params=pltpu.CompilerParams(
            use_tc_tiling_on_sc=False, needs_layout_passes=False),
        scratch_types=[
            pltpu.VMEM((NP,), jnp.int32),
            pltpu.VMEM((NP, 16), jnp.float32),
            pltpu.VMEM((NP, 16), jnp.float32),
            pltpu.SemaphoreType.DMA,
        ],
    )(labelt, mapping)

    parts = pl.pallas_call(
        _c_body,
        grid=(B // RB,),
        in_specs=[
            pl.BlockSpec((RB, N), lambda i: (i, 0)),
            pl.BlockSpec((NP, RB), lambda i: (0, i)),
        ],
        out_specs=pl.BlockSpec((1, 1, 1), lambda i: (i, 0, 0)),
        out_shape=jax.ShapeDtypeStruct((B // RB, 1, 1), jnp.float32),
    )(prediction, wint)

    return jnp.sum(parts) / jnp.float32(B)


# speedup vs baseline: 1.3107x; 1.0049x over previous
import functools

import jax
import jax.numpy as jnp
from jax import lax
from jax.experimental import pallas as pl
from jax.experimental.pallas import tpu as pltpu
from jax.experimental.pallas import tpu_sc as plsc

B = 1024
N = 3129
D = 768
NP = 3200
RB = 128
NB = NP // RB
NEG = -3.0e38

NC = 2
NS = 16
NW = NC * NS
ROWS_W = B // NW
CH = 640
NCH = NP // CH


def _a2_body(e_blk, e_full, map_ref, rec_ref):
    i = pl.program_id(0)

    @pl.when(i == 0)
    def _():
        e = e_full[...]
        g = lax.dot_general(e, e, (((0,), (0,)), ((), ())),
                            preferred_element_type=jnp.float32)
        h = lax.dot_general(e, g.astype(jnp.bfloat16),
                            (((1,), (0,)), ((), ())),
                            preferred_element_type=jnp.float32)
        n2 = jnp.sum(e.astype(jnp.float32) * h, axis=1)
        rec_ref[...] = (1.0 / jnp.sqrt(n2)).reshape(1, NP)

    s = lax.dot_general(e_blk[...], e_full[...], (((1,), (1,)), ((), ())),
                        preferred_element_type=jnp.float32)
    ref_ = rec_ref[...].reshape(NP)
    reb = rec_ref[0, pl.ds(i * RB, RB)]
    v = s * (reb[:, None] * ref_[None, :])
    col = lax.broadcasted_iota(jnp.int32, (RB, NP), 1)
    rowg = i * RB + lax.broadcasted_iota(jnp.int32, (RB, NP), 0)
    v = jnp.where(col == rowg, -2.0, v)
    v = jnp.where(col >= N, NEG, v)
    m = jnp.max(v, axis=1, keepdims=True)
    idx = jnp.min(jnp.where(v == m, col, NP), axis=1)
    map_ref[...] = idx.astype(jnp.int32).reshape(1, 1, RB)


def _c_body(x_ref, wt_ref, out_ref):
    x = x_ref[...]
    f = jnp.maximum(x, 0.0) + jnp.log(1.0 + jnp.exp(-jnp.abs(x)))
    w = jnp.transpose(wt_ref[...])[:, :N]
    out_ref[...] = (jnp.sum(f) - jnp.sum(x * w)).reshape(1, 1, 1)


def _sc_body(labt_hbm, map_hbm, wint_hbm, map_v, lab_v, win_v, sem):
    wid = lax.axis_index("s") * NC + lax.axis_index("c")
    pltpu.sync_copy(map_hbm, map_v)
    iot = lax.iota(jnp.int32, 16)
    zeros16 = jnp.zeros((16,), jnp.float32)

    out_dma = None
    for grp in range(ROWS_W // 16):
        r0 = wid * ROWS_W + grp * 16
        pltpu.sync_copy(labt_hbm.at[:, pl.ds(r0, 16)], lab_v)
        if out_dma is not None:
            out_dma.wait()

        def _initk(k, _):
            for kk in range(8):
                win_v[k * 8 + kk, :] = zeros16
            return 0
        lax.fori_loop(0, NP // 8, _initk, 0)

        UN = 16
        def _ph1(t, _):
            base = t * UN
            mch = map_v[pl.ds(base, 16)]
            for k in range(UN):
                gs = jnp.full((16,), base + k, jnp.int32)
                mvec = jnp.full((16,), mch[k], jnp.int32)
                lv = plsc.load_gather(lab_v, [gs, iot])
                plsc.store_scatter(win_v, [mvec, iot], lv, mask=lv != 0.0)
            return 0
        lax.fori_loop(0, N // UN, _ph1, 0)
        mch = map_v[pl.ds((N // UN) * UN, 16)]
        for k in range(N % UN):
            g = (N // UN) * UN + k
            gs = jnp.full((16,), g, jnp.int32)
            mvec = jnp.full((16,), mch[k], jnp.int32)
            lv = plsc.load_gather(lab_v, [gs, iot])
            plsc.store_scatter(win_v, [mvec, iot], lv, mask=lv != 0.0)

        out_dma = pltpu.async_copy(
            win_v, wint_hbm.at[:, pl.ds(r0, 16)], sem)
    out_dma.wait()


@jax.jit
def kernel(prediction, label, Embedding):
    ep = jnp.pad(Embedding, ((0, NP - N), (0, 0))).astype(jnp.bfloat16)
    mapping = pl.pallas_call(
        _a2_body,
        grid=(NB,),
        in_specs=[
            pl.BlockSpec((RB, D), lambda i: (i, 0)),
            pl.BlockSpec((NP, D), lambda i: (0, 0)),
        ],
        out_specs=pl.BlockSpec((1, 1, RB), lambda i: (i, 0, 0)),
        out_shape=jax.ShapeDtypeStruct((NB, 1, RB), jnp.int32),
        scratch_shapes=[pltpu.VMEM((1, NP), jnp.float32)],
    )(ep, ep).reshape(NP)

    labelt = jnp.pad(label.T, ((0, NP - N), (0, 0)))

    mesh = plsc.VectorSubcoreMesh(core_axis_name="c", subcore_axis_name="s")
    wint = pl.kernel(
        _sc_body,
        out_type=jax.ShapeDtypeStruct((NP, B), jnp.float32),
        mesh=mesh,
        compiler_params=pltpu.CompilerParams(
            use_tc_tiling_on_sc=False, needs_layout_passes=False),
        scratch_types=[
            pltpu.VMEM((NP,), jnp.int32),
            pltpu.VMEM((NP, 16), jnp.float32),
            pltpu.VMEM((NP, 16), jnp.float32),
            pltpu.SemaphoreType.DMA,
        ],
    )(labelt, mapping)

    parts = pl.pallas_call(
        _c_body,
        grid=(B // RB,),
        in_specs=[
            pl.BlockSpec((RB, N), lambda i: (i, 0)),
            pl.BlockSpec((NP, RB), lambda i: (0, i)),
        ],
        out_specs=pl.BlockSpec((1, 1, 1), lambda i: (i, 0, 0)),
        out_shape=jax.ShapeDtypeStruct((B // RB, 1, 1), jnp.float32),
    )(prediction, wint)

    return jnp.sum(parts) / jnp.float32(B)


# speedup vs baseline: 1.5954x; 1.2172x over previous
import functools

import jax
import jax.numpy as jnp
from jax import lax
from jax.experimental import pallas as pl
from jax.experimental.pallas import tpu as pltpu
from jax.experimental.pallas import tpu_sc as plsc

B = 1024
N = 3129
D = 768
NP = 3200
RB = 128
NB = NP // RB
NEG = -3.0e38

NC = 2
NS = 16
NW = NC * NS
ROWS_W = B // NW
CH = 640
NCH = NP // CH


def _a2_body(e_blk, e_full, map_ref, rec_ref):
    i = pl.program_id(0)

    @pl.when(i == 0)
    def _():
        e = e_full[...]
        g = lax.dot_general(e, e, (((0,), (0,)), ((), ())),
                            preferred_element_type=jnp.float32)
        h = lax.dot_general(e, g.astype(jnp.bfloat16),
                            (((1,), (0,)), ((), ())),
                            preferred_element_type=jnp.float32)
        n2 = jnp.sum(e.astype(jnp.float32) * h, axis=1)
        rec_ref[...] = (1.0 / jnp.sqrt(n2)).reshape(1, NP)

    s = lax.dot_general(e_blk[...], e_full[...], (((1,), (1,)), ((), ())),
                        preferred_element_type=jnp.float32)
    ref_ = rec_ref[...].reshape(NP)
    reb = rec_ref[0, pl.ds(i * RB, RB)]
    v = s * (reb[:, None] * ref_[None, :])
    col = lax.broadcasted_iota(jnp.int32, (RB, NP), 1)
    rowg = i * RB + lax.broadcasted_iota(jnp.int32, (RB, NP), 0)
    v = jnp.where(col == rowg, -2.0, v)
    v = jnp.where(col >= N, NEG, v)
    m = jnp.max(v, axis=1, keepdims=True)
    idx = jnp.min(jnp.where(v == m, col, NP), axis=1)
    map_ref[...] = idx.astype(jnp.int32).reshape(1, 1, RB)


def _c_body(x_ref, wt_ref, out_ref):
    x = x_ref[...]
    f = jnp.maximum(x, 0.0) + jnp.log(1.0 + jnp.exp(-jnp.abs(x)))
    w = jnp.transpose(wt_ref[...])[:, :N]
    out_ref[...] = (jnp.sum(f) - jnp.sum(x * w)).reshape(1, 1, 1)


def _sc_body(labt_hbm, map_hbm, wint_hbm, map_v, lab_v, win_v, sem):
    wid = lax.axis_index("s") * NC + lax.axis_index("c")
    pltpu.sync_copy(map_hbm, map_v)
    iot = lax.iota(jnp.int32, 16)
    zeros16 = jnp.zeros((16,), jnp.float32)

    out_dma = None
    for grp in range(ROWS_W // 16):
        r0 = wid * ROWS_W + grp * 16
        pltpu.sync_copy(labt_hbm.at[:, pl.ds(r0, 16)], lab_v)
        if out_dma is not None:
            out_dma.wait()

        def _initk(k, _):
            for kk in range(8):
                win_v[k * 8 + kk, :] = zeros16
            return 0
        lax.fori_loop(0, NP // 8, _initk, 0)

        UN = 8
        def _ph1(t, _):
            base = t * UN
            mch = map_v[pl.ds(base, 16)]
            mvecs = [jnp.full((16,), mch[k], jnp.int32) for k in range(UN)]
            lvs = [plsc.load_gather(
                lab_v, [jnp.full((16,), base + k, jnp.int32), iot])
                for k in range(UN)]
            msks = [plsc.bitcast(lv, jnp.int32) != 0 for lv in lvs]
            for k in range(UN):
                plsc.store_scatter(win_v, [mvecs[k], iot], lvs[k],
                                   mask=msks[k])
            return 0
        lax.fori_loop(0, N // UN, _ph1, 0)
        base = (N // UN) * UN
        mch = map_v[pl.ds(base, 16)]
        mvecs = [jnp.full((16,), mch[k], jnp.int32) for k in range(N % UN)]
        lvs = [plsc.load_gather(
            lab_v, [jnp.full((16,), base + k, jnp.int32), iot])
            for k in range(N % UN)]
        msks = [plsc.bitcast(lv, jnp.int32) != 0 for lv in lvs]
        for k in range(N % UN):
            plsc.store_scatter(win_v, [mvecs[k], iot], lvs[k], mask=msks[k])

        out_dma = pltpu.async_copy(
            win_v, wint_hbm.at[:, pl.ds(r0, 16)], sem)
    out_dma.wait()


@jax.jit
def kernel(prediction, label, Embedding):
    ep = jnp.pad(Embedding, ((0, NP - N), (0, 0))).astype(jnp.bfloat16)
    mapping = pl.pallas_call(
        _a2_body,
        grid=(NB,),
        in_specs=[
            pl.BlockSpec((RB, D), lambda i: (i, 0)),
            pl.BlockSpec((NP, D), lambda i: (0, 0)),
        ],
        out_specs=pl.BlockSpec((1, 1, RB), lambda i: (i, 0, 0)),
        out_shape=jax.ShapeDtypeStruct((NB, 1, RB), jnp.int32),
        scratch_shapes=[pltpu.VMEM((1, NP), jnp.float32)],
    )(ep, ep).reshape(NP)

    labelt = jnp.pad(label.T, ((0, NP - N), (0, 0)))

    mesh = plsc.VectorSubcoreMesh(core_axis_name="c", subcore_axis_name="s")
    wint = pl.kernel(
        _sc_body,
        out_type=jax.ShapeDtypeStruct((NP, B), jnp.float32),
        mesh=mesh,
        compiler_params=pltpu.CompilerParams(
            use_tc_tiling_on_sc=False, needs_layout_passes=False),
        scratch_types=[
            pltpu.VMEM((NP,), jnp.int32),
            pltpu.VMEM((NP, 16), jnp.float32),
            pltpu.VMEM((NP, 16), jnp.float32),
            pltpu.SemaphoreType.DMA,
        ],
    )(labelt, mapping)

    parts = pl.pallas_call(
        _c_body,
        grid=(B // RB,),
        in_specs=[
            pl.BlockSpec((RB, N), lambda i: (i, 0)),
            pl.BlockSpec((NP, RB), lambda i: (0, i)),
        ],
        out_specs=pl.BlockSpec((1, 1, 1), lambda i: (i, 0, 0)),
        out_shape=jax.ShapeDtypeStruct((B // RB, 1, 1), jnp.float32),
    )(prediction, wint)

    return jnp.sum(parts) / jnp.float32(B)


# speedup vs baseline: 1.6105x; 1.0095x over previous
import functools

import jax
import jax.numpy as jnp
from jax import lax
from jax.experimental import pallas as pl
from jax.experimental.pallas import tpu as pltpu
from jax.experimental.pallas import tpu_sc as plsc

B = 1024
N = 3129
D = 768
NP = 3200
RB = 128
NB = NP // RB
NEG = -3.0e38

NC = 2
NS = 16
NW = NC * NS
ROWS_W = B // NW
CH = 640
NCH = NP // CH


def _a2_body(e_blk, e_full, map_ref, rec_ref):
    i = pl.program_id(0)

    @pl.when(i == 0)
    def _():
        e = e_full[...]
        g = lax.dot_general(e, e, (((0,), (0,)), ((), ())),
                            preferred_element_type=jnp.float32)
        h = lax.dot_general(e, g.astype(jnp.bfloat16),
                            (((1,), (0,)), ((), ())),
                            preferred_element_type=jnp.float32)
        n2 = jnp.sum(e.astype(jnp.float32) * h, axis=1)
        rec_ref[...] = (1.0 / jnp.sqrt(n2)).reshape(1, NP)

    s = lax.dot_general(e_blk[...], e_full[...], (((1,), (1,)), ((), ())),
                        preferred_element_type=jnp.float32)
    ref_ = rec_ref[...].reshape(NP)
    reb = rec_ref[0, pl.ds(i * RB, RB)]
    v = s * (reb[:, None] * ref_[None, :])
    col = lax.broadcasted_iota(jnp.int32, (RB, NP), 1)
    rowg = i * RB + lax.broadcasted_iota(jnp.int32, (RB, NP), 0)
    v = jnp.where(col == rowg, -2.0, v)
    v = jnp.where(col >= N, NEG, v)
    m = jnp.max(v, axis=1, keepdims=True)
    idx = jnp.min(jnp.where(v == m, col, NP), axis=1)
    map_ref[...] = idx.astype(jnp.int32).reshape(1, 1, RB)


def _c_body(x_ref, wt_ref, out_ref):
    x = x_ref[...]
    f = jnp.maximum(x, 0.0) + jnp.log(1.0 + jnp.exp(-jnp.abs(x)))
    w = jnp.transpose(wt_ref[...])[:, :N]
    out_ref[...] = (jnp.sum(f) - jnp.sum(x * w)).reshape(1, 1, 1)


def _sc_body(labt_hbm, map_hbm, wint_hbm, map_v, lab_v, win_v, sem):
    wid = lax.axis_index("s") * NC + lax.axis_index("c")
    pltpu.sync_copy(map_hbm, map_v)
    iot = lax.iota(jnp.int32, 16)
    zeros16 = jnp.zeros((16,), jnp.float32)

    out_dma = None
    for grp in range(ROWS_W // 16):
        r0 = wid * ROWS_W + grp * 16
        pltpu.sync_copy(labt_hbm.at[:, pl.ds(r0, 16)], lab_v)
        if out_dma is not None:
            out_dma.wait()

        def _initk(k, _):
            for kk in range(8):
                win_v[k * 8 + kk, :] = zeros16
            return 0
        lax.fori_loop(0, NP // 8, _initk, 0)

        UN = 12
        def _ph1(t, _):
            base = t * UN
            mch = map_v[pl.ds(base, 16)]
            mvecs = [jnp.full((16,), mch[k], jnp.int32) for k in range(UN)]
            lvs = [plsc.load_gather(
                lab_v, [jnp.full((16,), base + k, jnp.int32), iot])
                for k in range(UN)]
            msks = [plsc.bitcast(lv, jnp.int32) != 0 for lv in lvs]
            for k in range(UN):
                plsc.store_scatter(win_v, [mvecs[k], iot], lvs[k],
                                   mask=msks[k])
            return 0
        lax.fori_loop(0, N // UN, _ph1, 0)
        base = (N // UN) * UN
        mch = map_v[pl.ds(base, 16)]
        mvecs = [jnp.full((16,), mch[k], jnp.int32) for k in range(N % UN)]
        lvs = [plsc.load_gather(
            lab_v, [jnp.full((16,), base + k, jnp.int32), iot])
            for k in range(N % UN)]
        msks = [plsc.bitcast(lv, jnp.int32) != 0 for lv in lvs]
        for k in range(N % UN):
            plsc.store_scatter(win_v, [mvecs[k], iot], lvs[k], mask=msks[k])

        out_dma = pltpu.async_copy(
            win_v, wint_hbm.at[:, pl.ds(r0, 16)], sem)
    out_dma.wait()


@jax.jit
def kernel(prediction, label, Embedding):
    ep = jnp.pad(Embedding, ((0, NP - N), (0, 0))).astype(jnp.bfloat16)
    mapping = pl.pallas_call(
        _a2_body,
        grid=(NB,),
        in_specs=[
            pl.BlockSpec((RB, D), lambda i: (i, 0)),
            pl.BlockSpec((NP, D), lambda i: (0, 0)),
        ],
        out_specs=pl.BlockSpec((1, 1, RB), lambda i: (i, 0, 0)),
        out_shape=jax.ShapeDtypeStruct((NB, 1, RB), jnp.int32),
        scratch_shapes=[pltpu.VMEM((1, NP), jnp.float32)],
    )(ep, ep).reshape(NP)

    labelt = jnp.pad(label.T, ((0, NP - N), (0, 0)))

    mesh = plsc.VectorSubcoreMesh(core_axis_name="c", subcore_axis_name="s")
    wint = pl.kernel(
        _sc_body,
        out_type=jax.ShapeDtypeStruct((NP, B), jnp.float32),
        mesh=mesh,
        compiler_params=pltpu.CompilerParams(
            use_tc_tiling_on_sc=False, needs_layout_passes=False),
        scratch_types=[
            pltpu.VMEM((NP,), jnp.int32),
            pltpu.VMEM((NP, 16), jnp.float32),
            pltpu.VMEM((NP, 16), jnp.float32),
            pltpu.SemaphoreType.DMA,
        ],
    )(labelt, mapping)

    parts = pl.pallas_call(
        _c_body,
        grid=(B // RB,),
        in_specs=[
            pl.BlockSpec((RB, N), lambda i: (i, 0)),
            pl.BlockSpec((NP, RB), lambda i: (0, i)),
        ],
        out_specs=pl.BlockSpec((1, 1, 1), lambda i: (i, 0, 0)),
        out_shape=jax.ShapeDtypeStruct((B // RB, 1, 1), jnp.float32),
    )(prediction, wint)

    return jnp.sum(parts) / jnp.float32(B)
